# baseline (speedup 1.0000x reference)
import functools

import jax
import jax.numpy as jnp
from jax import lax
from jax.experimental import pallas as pl
from jax.experimental.pallas import tpu as pltpu
from jax.experimental.pallas import tpu_sc as plsc

N = 10000
E = 320000
D_IN = 128
D_HID = 64
D_OUT = 128

NC = 2
NS = 16
NW = NC * NS
EPW = E // NW
K = 80
C = EPW // K
NP = 10240
RPT = NP // NS

_mesh = plsc.VectorSubcoreMesh(core_axis_name="c", subcore_axis_name="s")
_sc_params = pltpu.CompilerParams(use_tc_tiling_on_sc=False)


@functools.partial(
    pl.kernel,
    out_type=jax.ShapeDtypeStruct((NW, NP), jnp.float32),
    mesh=_mesh,
    scratch_types=[
        pltpu.VMEM((EPW,), jnp.int32),
        pltpu.VMEM((NP,), jnp.float32),
    ],
    compiler_params=pltpu.CompilerParams(
        use_tc_tiling_on_sc=False, needs_layout_passes=False
    ),
)
def _deg_sc(dst_hbm, out_hbm, dst_v, hist):
    c = lax.axis_index("c")
    s = lax.axis_index("s")
    wid = c * NS + s
    zero16 = jnp.zeros((16,), jnp.float32)
    ones16 = jnp.ones((16,), jnp.float32)

    def zb(i, carry):
        hist[pl.ds(i * 16, 16)] = zero16
        return carry

    lax.fori_loop(0, NP // 16, zb, 0)
    pltpu.sync_copy(dst_hbm.at[wid], dst_v)

    def body(r, carry):
        for q in range(5):
            ix = dst_v[pl.ds((r * 5 + q) * 16, 16)]
            plsc.addupdate_scatter(hist, [ix], ones16)
        return carry

    lax.fori_loop(0, EPW // 80, body, 0)
    pltpu.sync_copy(hist, out_hbm.at[wid])


@functools.partial(
    pl.kernel,
    out_type=jax.ShapeDtypeStruct((NC, NP, D_HID), jnp.float32),
    mesh=_mesh,
    scratch_types=[
        pltpu.VMEM((C, K), jnp.int32),
        pltpu.VMEM((C, K), jnp.int32),
        [pltpu.VMEM((K, D_HID), jnp.float32) for _ in range(5)],
        pltpu.VMEM_SHARED((NP, D_HID), jnp.float32),
        [pltpu.SemaphoreType.DMA for _ in range(5)],
        [pltpu.SemaphoreType.DMA for _ in range(5)],
        pltpu.SemaphoreType.DMA,
    ],
    compiler_params=_sc_params,
)
def _scatter_sc(src_hbm, dst_hbm, y_hbm, zeros_hbm, out_hbm,
                src_v, dst_v, bufs, acc, gsems, ssems, psem):
    c = lax.axis_index("c")
    s = lax.axis_index("s")
    wid = c * NS + s
    base = s * RPT
    pltpu.async_copy(zeros_hbm.at[pl.ds(base, RPT)], acc.at[pl.ds(base, RPT)],
                     psem)
    pltpu.async_copy(src_hbm.at[wid], src_v, gsems[0])
    pltpu.async_copy(dst_hbm.at[wid], dst_v, gsems[1])
    pltpu.make_async_copy(src_hbm.at[wid], src_v, gsems[0]).wait()
    pltpu.make_async_copy(dst_hbm.at[wid], dst_v, gsems[1]).wait()
    pltpu.make_async_copy(
        zeros_hbm.at[pl.ds(base, RPT)], acc.at[pl.ds(base, RPT)], psem).wait()
    plsc.subcore_barrier()

    U = 5
    for i in range(U):
        pltpu.async_copy(y_hbm.at[src_v.at[i]], bufs[i], gsems[i])

    def body(t, carry):
        for i in range(U):
            j = U * t + i
            pltpu.make_async_copy(y_hbm.at[src_v.at[j]], bufs[i], gsems[i]).wait()
            pltpu.async_copy(bufs[i], acc.at[dst_v.at[j]], ssems[i], add=True)
        for i in range(U):
            jn = U * t + U + i
            pltpu.make_async_copy(bufs[i], acc.at[dst_v.at[jn]], ssems[i]).wait()
            pltpu.async_copy(y_hbm.at[src_v.at[jn]], bufs[i], gsems[i])
        return carry

    lax.fori_loop(0, C // U - 1, body, 0)
    for i in range(U):
        j = C - U + i
        pltpu.make_async_copy(y_hbm.at[src_v.at[j]], bufs[i], gsems[i]).wait()
        pltpu.async_copy(bufs[i], acc.at[dst_v.at[j]], ssems[i], add=True)
    for i in range(U):
        pltpu.make_async_copy(bufs[i], acc.at[dst_v.at[C - U + i]], ssems[i]).wait()
    plsc.subcore_barrier()
    pltpu.sync_copy(acc.at[pl.ds(base, RPT)], out_hbm.at[c, pl.ds(base, RPT)])


HB = RPT // 4


@functools.partial(
    pl.kernel,
    out_type=(
        jax.ShapeDtypeStruct((NC, NP, D_HID), jnp.float32),
        jax.ShapeDtypeStruct((NC, NP, D_HID), jnp.float32),
    ),
    mesh=_mesh,
    scratch_types=[
        pltpu.VMEM((C, K), jnp.int32),
        pltpu.VMEM((C, K), jnp.int32),
        pltpu.VMEM((HB, D_HID), jnp.float32),
        pltpu.VMEM((HB, D_HID), jnp.float32),
        pltpu.VMEM((HB, D_HID), jnp.float32),
        pltpu.VMEM((HB, D_HID), jnp.float32),
        pltpu.VMEM((1, D_HID), jnp.float32),
        [pltpu.VMEM((K, D_HID), jnp.float32) for _ in range(5)],
        pltpu.VMEM_SHARED((NP, D_HID), jnp.float32),
        [pltpu.SemaphoreType.DMA for _ in range(5)],
        [pltpu.SemaphoreType.DMA for _ in range(5)],
        pltpu.SemaphoreType.DMA,
    ],
    compiler_params=_sc_params,
)
def _layer2_sc(src_hbm, dst_hbm, z1p_hbm, y1_hbm, dinvw_hbm, b1_hbm, zeros_hbm,
               y2_hbm, out_hbm,
               src_v, dst_v, za, zb, yv, dv, b1v,
               bufs, acc, gsems, ssems, psem):
    c = lax.axis_index("c")
    s = lax.axis_index("s")
    wid = c * NS + s
    base = s * RPT
    pltpu.async_copy(zeros_hbm.at[pl.ds(base, RPT)], acc.at[pl.ds(base, RPT)],
                     psem)
    pltpu.async_copy(src_hbm.at[wid], src_v, gsems[0])
    pltpu.async_copy(dst_hbm.at[wid], dst_v, gsems[1])
    pltpu.sync_copy(b1_hbm, b1v)
    b1q = [b1v[0, pl.ds(q * 16, 16)] for q in range(D_HID // 16)]

    for h in range(4):
        r0 = base + h * HB
        pltpu.async_copy(z1p_hbm.at[0, pl.ds(r0, HB)], za, ssems[0])
        pltpu.async_copy(z1p_hbm.at[1, pl.ds(r0, HB)], zb, ssems[1])
        pltpu.async_copy(y1_hbm.at[pl.ds(r0, HB)], yv, ssems[2])
        pltpu.async_copy(dinvw_hbm.at[pl.ds(r0, HB)], dv, ssems[3])
        pltpu.make_async_copy(z1p_hbm.at[0, pl.ds(r0, HB)], za, ssems[0]).wait()
        pltpu.make_async_copy(z1p_hbm.at[1, pl.ds(r0, HB)], zb, ssems[1]).wait()
        pltpu.make_async_copy(y1_hbm.at[pl.ds(r0, HB)], yv, ssems[2]).wait()
        pltpu.make_async_copy(dinvw_hbm.at[pl.ds(r0, HB)], dv, ssems[3]).wait()

        def ew(r, carry):
            for q in range(D_HID // 16):
                sl = pl.ds(q * 16, 16)
                d = dv[r, sl]
                v = za[r, sl] + zb[r, sl] + yv[r, sl]
                v = jnp.maximum(v * d + b1q[q], 0.0) * d
                za[r, sl] = v
            return carry

        lax.fori_loop(0, HB, ew, 0)
        pltpu.sync_copy(za, y2_hbm.at[c, pl.ds(r0, HB)])

    pltpu.make_async_copy(
        zeros_hbm.at[pl.ds(base, RPT)], acc.at[pl.ds(base, RPT)], psem).wait()
    pltpu.make_async_copy(src_hbm.at[wid], src_v, gsems[0]).wait()
    pltpu.make_async_copy(dst_hbm.at[wid], dst_v, gsems[1]).wait()
    plsc.subcore_barrier()

    y2c = y2_hbm.at[c]
    U = 5
    for i in range(U):
        pltpu.async_copy(y2c.at[src_v.at[i]], bufs[i], gsems[i])

    def body(t, carry):
        for i in range(U):
            j = U * t + i
            pltpu.make_async_copy(y2c.at[src_v.at[j]], bufs[i], gsems[i]).wait()
            pltpu.async_copy(bufs[i], acc.at[dst_v.at[j]], ssems[i], add=True)
        for i in range(U):
            jn = U * t + U + i
            pltpu.make_async_copy(bufs[i], acc.at[dst_v.at[jn]], ssems[i]).wait()
            pltpu.async_copy(y2c.at[src_v.at[jn]], bufs[i], gsems[i])
        return carry

    lax.fori_loop(0, C // U - 1, body, 0)
    for i in range(U):
        j = C - U + i
        pltpu.make_async_copy(y2c.at[src_v.at[j]], bufs[i], gsems[i]).wait()
        pltpu.async_copy(bufs[i], acc.at[dst_v.at[j]], ssems[i], add=True)
    for i in range(U):
        pltpu.make_async_copy(bufs[i], acc.at[dst_v.at[C - U + i]], ssems[i]).wait()
    plsc.subcore_barrier()
    pltpu.sync_copy(acc.at[pl.ds(base, RPT)], out_hbm.at[c, pl.ds(base, RPT)])


def _tc1_body(parts_ref, x_ref, w1_ref, dinv_ref, dinvw_ref, y1_ref):
    deg_col = lax.dot_general(
        parts_ref[...],
        jnp.ones((NW, 1), jnp.float32),
        (((0,), (0,)), ((), ())),
        preferred_element_type=jnp.float32,
    )
    dinv = lax.rsqrt(deg_col + 1.0)
    dinv_ref[...] = dinv
    dinvw_ref[...] = jnp.broadcast_to(dinv, (NP, D_HID))
    xw = jnp.dot(x_ref[...], w1_ref[...], preferred_element_type=jnp.float32)
    y1_ref[...] = dinv * xw


def _tc3_body(zp_ref, y2_ref, dinv_ref, w2_ref, b2_ref, out_ref):
    ah = dinv_ref[:N, :] * (
        zp_ref[0, :N, :] + zp_ref[1, :N, :] + y2_ref[0, :N, :]
    )
    out_ref[...] = (
        jnp.dot(ah, w2_ref[...], preferred_element_type=jnp.float32)
        + b2_ref[...]
    )


_tc1 = pl.pallas_call(
    _tc1_body,
    out_shape=(
        jax.ShapeDtypeStruct((NP, 1), jnp.float32),
        jax.ShapeDtypeStruct((NP, D_HID), jnp.float32),
        jax.ShapeDtypeStruct((NP, D_HID), jnp.float32),
    ),
)
_tc3 = pl.pallas_call(
    _tc3_body,
    out_shape=jax.ShapeDtypeStruct((N, D_OUT), jnp.float32),
)


def kernel(x, edge_index, W1, b1, W2, b2):
    src = edge_index[0].reshape(NW, C, K)
    dst = edge_index[1].reshape(NW, C, K)
    dst_flat = edge_index[1].reshape(NW, EPW)
    zeros_rows = jnp.zeros((NP, D_HID), jnp.float32)

    xp = jnp.pad(x, ((0, NP - N), (0, 0)))
    deg_parts = _deg_sc(dst_flat)
    dinv, dinvw, y1 = _tc1(deg_parts, xp, W1)
    z1_parts = _scatter_sc(src, dst, y1, zeros_rows)
    y2_copies, z2_parts = _layer2_sc(
        src, dst, z1_parts, y1, dinvw, b1.reshape(1, D_HID), zeros_rows)
    return _tc3(z2_parts, y2_copies, dinv, W2, b2.reshape(1, D_OUT))


# speedup vs baseline: 1.0490x; 1.0490x over previous
import functools

import jax
import jax.numpy as jnp
from jax import lax
from jax.experimental import pallas as pl
from jax.experimental.pallas import tpu as pltpu
from jax.experimental.pallas import tpu_sc as plsc

N = 10000
E = 320000
D_IN = 128
D_HID = 64
D_OUT = 128

NC = 2
NS = 16
NW = NC * NS
EPW = E // NW
K = 80
C = EPW // K
NP = 10240
RPT = NP // NS

_mesh = plsc.VectorSubcoreMesh(core_axis_name="c", subcore_axis_name="s")
_sc_params = pltpu.CompilerParams(use_tc_tiling_on_sc=False)


@functools.partial(
    pl.kernel,
    out_type=jax.ShapeDtypeStruct((NW, NP), jnp.float32),
    mesh=_mesh,
    scratch_types=[
        pltpu.VMEM((EPW,), jnp.int32),
        pltpu.VMEM((NP,), jnp.float32),
    ],
    compiler_params=pltpu.CompilerParams(
        use_tc_tiling_on_sc=False, needs_layout_passes=False
    ),
)
def _deg_sc(dst_hbm, out_hbm, dst_v, hist):
    c = lax.axis_index("c")
    s = lax.axis_index("s")
    wid = c * NS + s
    zero16 = jnp.zeros((16,), jnp.float32)
    ones16 = jnp.ones((16,), jnp.float32)

    def zb(i, carry):
        hist[pl.ds(i * 16, 16)] = zero16
        return carry

    lax.fori_loop(0, NP // 16, zb, 0)
    pltpu.sync_copy(dst_hbm.at[wid], dst_v)

    def body(r, carry):
        for q in range(5):
            ix = dst_v[pl.ds((r * 5 + q) * 16, 16)]
            plsc.addupdate_scatter(hist, [ix], ones16)
        return carry

    lax.fori_loop(0, EPW // 80, body, 0)
    pltpu.sync_copy(hist, out_hbm.at[wid])


@functools.partial(
    pl.kernel,
    out_type=jax.ShapeDtypeStruct((NC, NP, D_HID), jnp.float32),
    mesh=_mesh,
    scratch_types=[
        pltpu.VMEM((C, K), jnp.int32),
        pltpu.VMEM((C, K), jnp.int32),
        [pltpu.VMEM((K, D_HID), jnp.float32) for _ in range(5)],
        pltpu.VMEM_SHARED((NP, D_HID), jnp.float32),
        [pltpu.SemaphoreType.DMA for _ in range(5)],
        [pltpu.SemaphoreType.DMA for _ in range(5)],
        pltpu.SemaphoreType.DMA,
    ],
    compiler_params=_sc_params,
)
def _scatter_sc(src_hbm, dst_hbm, y_hbm, zeros_hbm, out_hbm,
                src_v, dst_v, bufs, acc, gsems, ssems, psem):
    c = lax.axis_index("c")
    s = lax.axis_index("s")
    wid = c * NS + s
    base = s * RPT

    @pl.when(c == 0)
    def _():
        pltpu.async_copy(y_hbm.at[pl.ds(base, RPT)], acc.at[pl.ds(base, RPT)],
                         psem)

    @pl.when(c == 1)
    def _():
        pltpu.async_copy(zeros_hbm.at[pl.ds(base, RPT)],
                         acc.at[pl.ds(base, RPT)], psem)

    pltpu.async_copy(src_hbm.at[wid], src_v, gsems[0])
    pltpu.async_copy(dst_hbm.at[wid], dst_v, gsems[1])
    pltpu.make_async_copy(src_hbm.at[wid], src_v, gsems[0]).wait()
    pltpu.make_async_copy(dst_hbm.at[wid], dst_v, gsems[1]).wait()
    pltpu.make_async_copy(
        zeros_hbm.at[pl.ds(base, RPT)], acc.at[pl.ds(base, RPT)], psem).wait()
    plsc.subcore_barrier()

    U = 5
    for i in range(U):
        pltpu.async_copy(y_hbm.at[src_v.at[i]], bufs[i], gsems[i])

    def body(t, carry):
        for i in range(U):
            j = U * t + i
            pltpu.make_async_copy(y_hbm.at[src_v.at[j]], bufs[i], gsems[i]).wait()
            pltpu.async_copy(bufs[i], acc.at[dst_v.at[j]], ssems[i], add=True)
        for i in range(U):
            jn = U * t + U + i
            pltpu.make_async_copy(bufs[i], acc.at[dst_v.at[jn]], ssems[i]).wait()
            pltpu.async_copy(y_hbm.at[src_v.at[jn]], bufs[i], gsems[i])
        return carry

    lax.fori_loop(0, C // U - 1, body, 0)
    for i in range(U):
        j = C - U + i
        pltpu.make_async_copy(y_hbm.at[src_v.at[j]], bufs[i], gsems[i]).wait()
        pltpu.async_copy(bufs[i], acc.at[dst_v.at[j]], ssems[i], add=True)
    for i in range(U):
        pltpu.make_async_copy(bufs[i], acc.at[dst_v.at[C - U + i]], ssems[i]).wait()
    plsc.subcore_barrier()
    pltpu.sync_copy(acc.at[pl.ds(base, RPT)], out_hbm.at[c, pl.ds(base, RPT)])


HB = RPT // 4


@functools.partial(
    pl.kernel,
    out_type=(
        jax.ShapeDtypeStruct((NC, NP, D_HID), jnp.float32),
        jax.ShapeDtypeStruct((NC, NP, D_HID), jnp.float32),
    ),
    mesh=_mesh,
    scratch_types=[
        pltpu.VMEM((C, K), jnp.int32),
        pltpu.VMEM((C, K), jnp.int32),
        pltpu.VMEM((HB, D_HID), jnp.float32),
        pltpu.VMEM((HB, D_HID), jnp.float32),
        pltpu.VMEM((HB, D_HID), jnp.float32),
        pltpu.VMEM((1, D_HID), jnp.float32),
        [pltpu.VMEM((K, D_HID), jnp.float32) for _ in range(5)],
        pltpu.VMEM_SHARED((NP, D_HID), jnp.float32),
        [pltpu.SemaphoreType.DMA for _ in range(5)],
        [pltpu.SemaphoreType.DMA for _ in range(5)],
        pltpu.SemaphoreType.DMA,
    ],
    compiler_params=_sc_params,
)
def _layer2_sc(src_hbm, dst_hbm, z1p_hbm, dinvw_hbm, b1_hbm, zeros_hbm,
               y2_hbm, out_hbm,
               src_v, dst_v, za, zb, dv, b1v,
               bufs, acc, gsems, ssems, psem):
    c = lax.axis_index("c")
    s = lax.axis_index("s")
    wid = c * NS + s
    base = s * RPT

    @pl.when(c == 1)
    def _():
        pltpu.async_copy(zeros_hbm.at[pl.ds(base, RPT)],
                         acc.at[pl.ds(base, RPT)], psem)

    pltpu.async_copy(src_hbm.at[wid], src_v, gsems[0])
    pltpu.async_copy(dst_hbm.at[wid], dst_v, gsems[1])
    pltpu.sync_copy(b1_hbm, b1v)
    b1q = [b1v[0, pl.ds(q * 16, 16)] for q in range(D_HID // 16)]

    for h in range(4):
        r0 = base + h * HB
        pltpu.async_copy(z1p_hbm.at[0, pl.ds(r0, HB)], za, ssems[0])
        pltpu.async_copy(z1p_hbm.at[1, pl.ds(r0, HB)], zb, ssems[1])
        pltpu.async_copy(dinvw_hbm.at[pl.ds(r0, HB)], dv, ssems[3])
        pltpu.make_async_copy(z1p_hbm.at[0, pl.ds(r0, HB)], za, ssems[0]).wait()
        pltpu.make_async_copy(z1p_hbm.at[1, pl.ds(r0, HB)], zb, ssems[1]).wait()
        pltpu.make_async_copy(dinvw_hbm.at[pl.ds(r0, HB)], dv, ssems[3]).wait()

        def ew(r, carry):
            for q in range(D_HID // 16):
                sl = pl.ds(q * 16, 16)
                d = dv[r, sl]
                v = za[r, sl] + zb[r, sl]
                v = jnp.maximum(v * d + b1q[q], 0.0) * d
                za[r, sl] = v
            return carry

        lax.fori_loop(0, HB, ew, 0)
        pltpu.sync_copy(za, y2_hbm.at[c, pl.ds(r0, HB)])

        @pl.when(c == 0)
        def _():
            pltpu.sync_copy(za, acc.at[pl.ds(r0, HB)])

    @pl.when(c == 1)
    def _():
        pltpu.make_async_copy(
            zeros_hbm.at[pl.ds(base, RPT)], acc.at[pl.ds(base, RPT)],
            psem).wait()

    pltpu.make_async_copy(src_hbm.at[wid], src_v, gsems[0]).wait()
    pltpu.make_async_copy(dst_hbm.at[wid], dst_v, gsems[1]).wait()
    plsc.subcore_barrier()

    y2c = y2_hbm.at[c]
    U = 5
    for i in range(U):
        pltpu.async_copy(y2c.at[src_v.at[i]], bufs[i], gsems[i])

    def body(t, carry):
        for i in range(U):
            j = U * t + i
            pltpu.make_async_copy(y2c.at[src_v.at[j]], bufs[i], gsems[i]).wait()
            pltpu.async_copy(bufs[i], acc.at[dst_v.at[j]], ssems[i], add=True)
        for i in range(U):
            jn = U * t + U + i
            pltpu.make_async_copy(bufs[i], acc.at[dst_v.at[jn]], ssems[i]).wait()
            pltpu.async_copy(y2c.at[src_v.at[jn]], bufs[i], gsems[i])
        return carry

    lax.fori_loop(0, C // U - 1, body, 0)
    for i in range(U):
        j = C - U + i
        pltpu.make_async_copy(y2c.at[src_v.at[j]], bufs[i], gsems[i]).wait()
        pltpu.async_copy(bufs[i], acc.at[dst_v.at[j]], ssems[i], add=True)
    for i in range(U):
        pltpu.make_async_copy(bufs[i], acc.at[dst_v.at[C - U + i]], ssems[i]).wait()
    plsc.subcore_barrier()
    pltpu.sync_copy(acc.at[pl.ds(base, RPT)], out_hbm.at[c, pl.ds(base, RPT)])


def _tc1_body(parts_ref, x_ref, w1_ref, dinv_ref, dinvw_ref, y1_ref):
    deg_col = lax.dot_general(
        parts_ref[...],
        jnp.ones((NW, 1), jnp.float32),
        (((0,), (0,)), ((), ())),
        preferred_element_type=jnp.float32,
    )
    dinv = lax.rsqrt(deg_col + 1.0)
    dinv_ref[...] = dinv
    dinvw_ref[...] = jnp.broadcast_to(dinv, (NP, D_HID))
    xw = jnp.dot(x_ref[...], w1_ref[...], preferred_element_type=jnp.float32)
    y1_ref[...] = dinv * xw


def _tc3_body(zp_ref, dinv_ref, w2_ref, b2_ref, out_ref):
    ah = dinv_ref[:N, :] * (zp_ref[0, :N, :] + zp_ref[1, :N, :])
    out_ref[...] = (
        jnp.dot(ah, w2_ref[...], preferred_element_type=jnp.float32)
        + b2_ref[...]
    )


_tc1 = pl.pallas_call(
    _tc1_body,
    out_shape=(
        jax.ShapeDtypeStruct((NP, 1), jnp.float32),
        jax.ShapeDtypeStruct((NP, D_HID), jnp.float32),
        jax.ShapeDtypeStruct((NP, D_HID), jnp.float32),
    ),
)
_tc3 = pl.pallas_call(
    _tc3_body,
    out_shape=jax.ShapeDtypeStruct((N, D_OUT), jnp.float32),
)


def kernel(x, edge_index, W1, b1, W2, b2):
    src = edge_index[0].reshape(NW, C, K)
    dst = edge_index[1].reshape(NW, C, K)
    dst_flat = edge_index[1].reshape(NW, EPW)
    zeros_rows = jnp.zeros((NP, D_HID), jnp.float32)

    xp = jnp.pad(x, ((0, NP - N), (0, 0)))
    deg_parts = _deg_sc(dst_flat)
    dinv, dinvw, y1 = _tc1(deg_parts, xp, W1)
    z1_parts = _scatter_sc(src, dst, y1, zeros_rows)
    y2_copies, z2_parts = _layer2_sc(
        src, dst, z1_parts, dinvw, b1.reshape(1, D_HID), zeros_rows)
    del y2_copies
    return _tc3(z2_parts, dinv, W2, b2.reshape(1, D_OUT))


# speedup vs baseline: 1.0727x; 1.0226x over previous
import functools

import jax
import jax.numpy as jnp
from jax import lax
from jax.experimental import pallas as pl
from jax.experimental.pallas import tpu as pltpu
from jax.experimental.pallas import tpu_sc as plsc

N = 10000
E = 320000
D_IN = 128
D_HID = 64
D_OUT = 128

NC = 2
NS = 16
NW = NC * NS
EPW = E // NW
K = 80
C = EPW // K
NP = 10240
RPT = NP // NS

_mesh = plsc.VectorSubcoreMesh(core_axis_name="c", subcore_axis_name="s")
_sc_params = pltpu.CompilerParams(use_tc_tiling_on_sc=False)


@functools.partial(
    pl.kernel,
    out_type=jax.ShapeDtypeStruct((NW, NP), jnp.float32),
    mesh=_mesh,
    scratch_types=[
        pltpu.VMEM((EPW,), jnp.int32),
        pltpu.VMEM((NP,), jnp.float32),
    ],
    compiler_params=pltpu.CompilerParams(
        use_tc_tiling_on_sc=False, needs_layout_passes=False
    ),
)
def _deg_sc(dst_hbm, out_hbm, dst_v, hist):
    c = lax.axis_index("c")
    s = lax.axis_index("s")
    wid = c * NS + s
    zero16 = jnp.zeros((16,), jnp.float32)
    ones16 = jnp.ones((16,), jnp.float32)

    def zb(i, carry):
        hist[pl.ds(i * 16, 16)] = zero16
        return carry

    lax.fori_loop(0, NP // 16, zb, 0)
    pltpu.sync_copy(dst_hbm.at[wid], dst_v)

    def body(r, carry):
        for q in range(5):
            ix = dst_v[pl.ds((r * 5 + q) * 16, 16)]
            plsc.addupdate_scatter(hist, [ix], ones16)
        return carry

    lax.fori_loop(0, EPW // 80, body, 0)
    pltpu.sync_copy(hist, out_hbm.at[wid])


@functools.partial(
    pl.kernel,
    out_type=jax.ShapeDtypeStruct((NC, NP, D_HID), jnp.float32),
    mesh=_mesh,
    scratch_types=[
        pltpu.VMEM((C, K), jnp.int32),
        pltpu.VMEM((C, K), jnp.int32),
        [pltpu.VMEM((K, D_HID), jnp.float32) for _ in range(5)],
        pltpu.VMEM_SHARED((NP, D_HID), jnp.float32),
        [pltpu.SemaphoreType.DMA for _ in range(5)],
        [pltpu.SemaphoreType.DMA for _ in range(5)],
        pltpu.SemaphoreType.DMA,
    ],
    compiler_params=_sc_params,
)
def _scatter_sc(src_hbm, dst_hbm, y_hbm, zeros_hbm, out_hbm,
                src_v, dst_v, bufs, acc, gsems, ssems, psem):
    c = lax.axis_index("c")
    s = lax.axis_index("s")
    wid = c * NS + s
    base = s * RPT

    @pl.when(c == 0)
    def _():
        pltpu.async_copy(y_hbm.at[pl.ds(base, RPT)], acc.at[pl.ds(base, RPT)],
                         psem)

    @pl.when(c == 1)
    def _():
        pltpu.async_copy(zeros_hbm.at[pl.ds(base, RPT)],
                         acc.at[pl.ds(base, RPT)], psem)

    pltpu.async_copy(src_hbm.at[wid], src_v, gsems[0])
    pltpu.async_copy(dst_hbm.at[wid], dst_v, gsems[1])
    pltpu.make_async_copy(src_hbm.at[wid], src_v, gsems[0]).wait()
    pltpu.make_async_copy(dst_hbm.at[wid], dst_v, gsems[1]).wait()
    pltpu.make_async_copy(
        zeros_hbm.at[pl.ds(base, RPT)], acc.at[pl.ds(base, RPT)], psem).wait()
    plsc.subcore_barrier()

    U = 5
    for i in range(U):
        pltpu.async_copy(y_hbm.at[src_v.at[i]], bufs[i], gsems[i])

    def body(t, carry):
        for i in range(U):
            j = U * t + i
            pltpu.make_async_copy(y_hbm.at[src_v.at[j]], bufs[i], gsems[i]).wait()
            pltpu.async_copy(bufs[i], acc.at[dst_v.at[j]], ssems[i], add=True)
        for i in range(U):
            jn = U * t + U + i
            pltpu.make_async_copy(bufs[i], acc.at[dst_v.at[jn]], ssems[i]).wait()
            pltpu.async_copy(y_hbm.at[src_v.at[jn]], bufs[i], gsems[i])
        return carry

    lax.fori_loop(0, C // U - 1, body, 0)
    for i in range(U):
        j = C - U + i
        pltpu.make_async_copy(y_hbm.at[src_v.at[j]], bufs[i], gsems[i]).wait()
        pltpu.async_copy(bufs[i], acc.at[dst_v.at[j]], ssems[i], add=True)
    for i in range(U):
        pltpu.make_async_copy(bufs[i], acc.at[dst_v.at[C - U + i]], ssems[i]).wait()
    plsc.subcore_barrier()
    pltpu.sync_copy(acc.at[pl.ds(base, RPT)], out_hbm.at[c, pl.ds(base, RPT)])


HB = RPT // 5


@functools.partial(
    pl.kernel,
    out_type=(
        jax.ShapeDtypeStruct((NC, NP, D_HID), jnp.float32),
        jax.ShapeDtypeStruct((NC, NP, D_HID), jnp.float32),
    ),
    mesh=_mesh,
    scratch_types=[
        pltpu.VMEM((C, K), jnp.int32),
        pltpu.VMEM((C, K), jnp.int32),
        [pltpu.VMEM((HB, D_HID), jnp.float32) for _ in range(2)],
        [pltpu.VMEM((HB, D_HID), jnp.float32) for _ in range(2)],
        pltpu.VMEM((HB, D_HID), jnp.float32),
        pltpu.VMEM((1, D_HID), jnp.float32),
        [pltpu.VMEM((K, D_HID), jnp.float32) for _ in range(5)],
        pltpu.VMEM_SHARED((NP, D_HID), jnp.float32),
        [pltpu.SemaphoreType.DMA for _ in range(5)],
        [pltpu.SemaphoreType.DMA for _ in range(5)],
        pltpu.SemaphoreType.DMA,
    ],
    compiler_params=_sc_params,
)
def _layer2_sc(src_hbm, dst_hbm, z1p_hbm, dinvw_hbm, b1_hbm, zeros_hbm,
               y2_hbm, out_hbm,
               src_v, dst_v, zas, zbs, dvs, b1v,
               bufs, acc, gsems, ssems, psem):
    c = lax.axis_index("c")
    s = lax.axis_index("s")
    wid = c * NS + s
    base = s * RPT

    @pl.when(c == 1)
    def _():
        pltpu.async_copy(zeros_hbm.at[pl.ds(base, RPT)],
                         acc.at[pl.ds(base, RPT)], psem)

    pltpu.async_copy(src_hbm.at[wid], src_v, gsems[0])
    pltpu.async_copy(dst_hbm.at[wid], dst_v, gsems[1])
    pltpu.sync_copy(b1_hbm, b1v)
    b1q = [b1v[0, pl.ds(q * 16, 16)] for q in range(D_HID // 16)]

    def _load(h):
        r0 = base + h * HB
        p = h % 2
        pltpu.async_copy(z1p_hbm.at[0, pl.ds(r0, HB)], zas[p], ssems[p])
        pltpu.async_copy(z1p_hbm.at[1, pl.ds(r0, HB)], zbs[p], ssems[2 + p])

    _load(0)
    for h in range(5):
        r0 = base + h * HB
        p = h % 2
        za, zb, dvp = zas[p], zbs[p], dvs
        pltpu.sync_copy(dinvw_hbm.at[pl.ds(r0, HB)], dvp)
        pltpu.make_async_copy(z1p_hbm.at[0, pl.ds(r0, HB)], za,
                              ssems[p]).wait()
        pltpu.make_async_copy(z1p_hbm.at[1, pl.ds(r0, HB)], zb,
                              ssems[2 + p]).wait()
        if h < 4:
            if h >= 1:
                pltpu.make_async_copy(zas[1 - p],
                                      y2_hbm.at[c, pl.ds(r0 - HB, HB)],
                                      ssems[4]).wait()

                @pl.when(c == 0)
                def _():
                    pltpu.make_async_copy(zas[1 - p],
                                          acc.at[pl.ds(r0 - HB, HB)],
                                          gsems[4]).wait()

            _load(h + 1)

        def ew(r, carry):
            for q in range(D_HID // 16):
                sl = pl.ds(q * 16, 16)
                d = dvp[r, sl]
                v = za[r, sl] + zb[r, sl]
                v = jnp.maximum(v * d + b1q[q], 0.0) * d
                za[r, sl] = v
            return carry

        lax.fori_loop(0, HB, ew, 0)
        pltpu.async_copy(za, y2_hbm.at[c, pl.ds(r0, HB)], ssems[4])

        @pl.when(c == 0)
        def _():
            pltpu.async_copy(za, acc.at[pl.ds(r0, HB)], gsems[4])

    for h in (3, 4):
        r0 = base + h * HB
        pltpu.make_async_copy(zas[h % 2], y2_hbm.at[c, pl.ds(r0, HB)],
                              ssems[4]).wait()

        @pl.when(c == 0)
        def _():
            pltpu.make_async_copy(zas[h % 2], acc.at[pl.ds(r0, HB)],
                                  gsems[4]).wait()

    @pl.when(c == 1)
    def _():
        pltpu.make_async_copy(
            zeros_hbm.at[pl.ds(base, RPT)], acc.at[pl.ds(base, RPT)],
            psem).wait()

    pltpu.make_async_copy(src_hbm.at[wid], src_v, gsems[0]).wait()
    pltpu.make_async_copy(dst_hbm.at[wid], dst_v, gsems[1]).wait()
    plsc.subcore_barrier()

    y2c = y2_hbm.at[c]
    U = 5
    for i in range(U):
        pltpu.async_copy(y2c.at[src_v.at[i]], bufs[i], gsems[i])

    def body(t, carry):
        for i in range(U):
            j = U * t + i
            pltpu.make_async_copy(y2c.at[src_v.at[j]], bufs[i], gsems[i]).wait()
            pltpu.async_copy(bufs[i], acc.at[dst_v.at[j]], ssems[i], add=True)
        for i in range(U):
            jn = U * t + U + i
            pltpu.make_async_copy(bufs[i], acc.at[dst_v.at[jn]], ssems[i]).wait()
            pltpu.async_copy(y2c.at[src_v.at[jn]], bufs[i], gsems[i])
        return carry

    lax.fori_loop(0, C // U - 1, body, 0)
    for i in range(U):
        j = C - U + i
        pltpu.make_async_copy(y2c.at[src_v.at[j]], bufs[i], gsems[i]).wait()
        pltpu.async_copy(bufs[i], acc.at[dst_v.at[j]], ssems[i], add=True)
    for i in range(U):
        pltpu.make_async_copy(bufs[i], acc.at[dst_v.at[C - U + i]], ssems[i]).wait()
    plsc.subcore_barrier()
    pltpu.sync_copy(acc.at[pl.ds(base, RPT)], out_hbm.at[c, pl.ds(base, RPT)])


def _tc1_body(parts_ref, x_ref, w1_ref, dinv_ref, dinvw_ref, y1_ref):
    deg_col = lax.dot_general(
        parts_ref[...],
        jnp.ones((NW, 1), jnp.float32),
        (((0,), (0,)), ((), ())),
        preferred_element_type=jnp.float32,
    )
    dinv = lax.rsqrt(deg_col + 1.0)
    dinv_ref[...] = dinv
    dinvw_ref[...] = jnp.broadcast_to(dinv, (NP, D_HID))
    xw = jnp.dot(x_ref[...], w1_ref[...], preferred_element_type=jnp.float32)
    y1_ref[...] = dinv * xw


def _tc3_body(zp_ref, dinv_ref, w2_ref, b2_ref, out_ref):
    ah = dinv_ref[:N, :] * (zp_ref[0, :N, :] + zp_ref[1, :N, :])
    out_ref[...] = (
        jnp.dot(ah, w2_ref[...], preferred_element_type=jnp.float32)
        + b2_ref[...]
    )


_tc1 = pl.pallas_call(
    _tc1_body,
    out_shape=(
        jax.ShapeDtypeStruct((NP, 1), jnp.float32),
        jax.ShapeDtypeStruct((NP, D_HID), jnp.float32),
        jax.ShapeDtypeStruct((NP, D_HID), jnp.float32),
    ),
)
_tc3 = pl.pallas_call(
    _tc3_body,
    out_shape=jax.ShapeDtypeStruct((N, D_OUT), jnp.float32),
)


def kernel(x, edge_index, W1, b1, W2, b2):
    src = edge_index[0].reshape(NW, C, K)
    dst = edge_index[1].reshape(NW, C, K)
    dst_flat = edge_index[1].reshape(NW, EPW)
    zeros_rows = jnp.zeros((NP, D_HID), jnp.float32)

    xp = jnp.pad(x, ((0, NP - N), (0, 0)))
    deg_parts = _deg_sc(dst_flat)
    dinv, dinvw, y1 = _tc1(deg_parts, xp, W1)
    z1_parts = _scatter_sc(src, dst, y1, zeros_rows)
    y2_copies, z2_parts = _layer2_sc(
        src, dst, z1_parts, dinvw, b1.reshape(1, D_HID), zeros_rows)
    del y2_copies
    return _tc3(z2_parts, dinv, W2, b2.reshape(1, D_OUT))
